# baseline (speedup 1.0000x reference)
import functools

import jax
import jax.numpy as jnp
from jax.experimental import pallas as pl
from jax.experimental.pallas import tpu as pltpu
from jax.experimental.pallas import tpu_sc as plsc

B, D_IN_ = 4096, 3072
H1_, H2_ = 1024, 512
D_EMB_, K_ = 256, 8192

_BM1 = 256
_BM3 = 256


def _rowsum_sq_lanes(v):
    sq = v * v
    s1 = sq[:, 0:128] + sq[:, 128:256]
    acc = s1[:, 0:8]
    for j in range(1, 16):
        acc = acc + s1[:, 8 * j:8 * (j + 1)]
    a4 = acc[:, 0:4] + acc[:, 4:8]
    a2 = a4[:, 0:2] + a4[:, 2:4]
    return a2[:, 0:1] + a2[:, 1:2]


def _rowsum_sq_sublanes(vT):
    sq = vT * vT
    s1 = sq[0:128, :] + sq[128:256, :]
    acc = s1[0:8, :]
    for j in range(1, 16):
        acc = acc + s1[8 * j:8 * (j + 1), :]
    a4 = acc[0:4, :] + acc[4:8, :]
    a2 = a4[0:2, :] + a4[2:4, :]
    return a2[0:1, :] + a2[1:2, :]


def _enc_vq_body(x_ref, we1_ref, be1_ref, we2_ref, be2_ref, we3_ref, be3_ref,
                 cb_ref, cbt_ref, z_ref, idx_ref, bv_ref):
    f32 = jnp.float32

    @pl.when(pl.program_id(0) == 0)
    def _():
        bv_ref[...] = _rowsum_sq_sublanes(cbt_ref[...])

    h1 = jnp.maximum(
        jax.lax.dot_general(x_ref[...], we1_ref[...], (((1,), (0,)), ((), ())),
                            preferred_element_type=f32) + be1_ref[...], 0.0)
    h2 = jnp.maximum(
        jax.lax.dot_general(h1, we2_ref[...], (((1,), (0,)), ((), ())),
                            preferred_element_type=f32) + be2_ref[...], 0.0)
    z = jax.lax.dot_general(h2, we3_ref[...], (((1,), (0,)), ((), ())),
                            preferred_element_type=f32) + be3_ref[...]
    a = _rowsum_sq_lanes(z)
    m2 = jax.lax.dot_general(z * 2.0, cb_ref[...], (((1,), (1,)), ((), ())),
                             preferred_element_type=f32)
    dist = (a + bv_ref[...]) - m2
    dmin = jnp.min(dist, axis=1, keepdims=True)
    iota = jax.lax.broadcasted_iota(jnp.int32, dist.shape, 1)
    idx = jnp.min(jnp.where(dist == dmin, iota, jnp.int32(K_)),
                  axis=1, keepdims=True)
    z_ref[...] = z
    idx_ref[...] = jnp.broadcast_to(idx, (idx.shape[0], 128))


def _enc_vq(x, We1, be1, We2, be2, We3, be3, cb, cbT):
    bm = _BM1
    grid = (x.shape[0] // bm,)
    const = lambda shape: pl.BlockSpec(shape, lambda i: (0,) * len(shape))
    return pl.pallas_call(
        _enc_vq_body,
        grid=grid,
        in_specs=[
            pl.BlockSpec((bm, D_IN_), lambda i: (i, 0)),
            const((D_IN_, H1_)),
            const((1, H1_)),
            const((H1_, H2_)),
            const((1, H2_)),
            const((H2_, D_EMB_)),
            const((1, D_EMB_)),
            const((K_, D_EMB_)),
            const((D_EMB_, K_)),
        ],
        out_specs=[
            pl.BlockSpec((bm, D_EMB_), lambda i: (i, 0)),
            pl.BlockSpec((bm, 128), lambda i: (i, 0)),
        ],
        out_shape=[
            jax.ShapeDtypeStruct((x.shape[0], D_EMB_), jnp.float32),
            jax.ShapeDtypeStruct((x.shape[0], 128), jnp.int32),
        ],
        scratch_shapes=[pltpu.VMEM((1, K_), jnp.float32)],
    )(x, We1, be1, We2, be2, We3, be3, cb, cbT)


def _sc_gather(cb, idx_row):
    mesh = plsc.VectorSubcoreMesh(core_axis_name="c", subcore_axis_name="s")
    win = 128
    n = idx_row.shape[1]

    @functools.partial(
        pl.kernel,
        out_type=jax.ShapeDtypeStruct((n, cb.shape[1]), cb.dtype),
        mesh=mesh)
    def k(cb_hbm, i_hbm, o_hbm):
        def body(i_vmem, o_vmem):
            pltpu.sync_copy(cb_hbm.at[i_vmem.at[0]], o_vmem)

        pltpu.emit_pipeline(
            body,
            grid=(n // win,),
            in_specs=[pl.BlockSpec((1, win), index_map=lambda i: (0, i))],
            out_specs=[pl.BlockSpec((win, cb.shape[1]),
                                    index_map=lambda i: (i, 0))],
            core_axis_name=("c", "s"),
            dimension_semantics=(pltpu.PARALLEL,),
        )(i_hbm, o_hbm)

    return k(cb, idx_row)


def _dec_body(z_ref, q_ref, wd1_ref, bd1_ref, wd2_ref, bd2_ref, wd3_ref,
              bd3_ref, zq_ref, recon_ref, lp_ref):
    f32 = jnp.float32
    bf16 = jnp.bfloat16
    z = z_ref[...]
    q = q_ref[...]
    qz = q - z
    zq = z + qz
    lp = jnp.sum(qz * qz)
    d1 = jnp.maximum(
        jax.lax.dot_general(zq.astype(bf16), wd1_ref[...],
                            (((1,), (0,)), ((), ())),
                            preferred_element_type=f32) + bd1_ref[...], 0.0)
    d2 = jnp.maximum(
        jax.lax.dot_general(d1.astype(bf16), wd2_ref[...],
                            (((1,), (0,)), ((), ())),
                            preferred_element_type=f32) + bd2_ref[...], 0.0)
    logits = jax.lax.dot_general(d2.astype(bf16), wd3_ref[...],
                                 (((1,), (0,)), ((), ())),
                                 preferred_element_type=f32) + bd3_ref[...]
    zq_ref[...] = zq
    recon_ref[...] = jax.nn.sigmoid(logits)
    lp_ref[...] = jnp.broadcast_to(lp.reshape(1, 1, 1), (1, 8, 128))


def _decode(z, q, Wd1b, bd1, Wd2b, bd2, Wd3b, bd3):
    bm = _BM3
    g = z.shape[0] // bm
    const = lambda shape: pl.BlockSpec(shape, lambda i: (0,) * len(shape))
    return pl.pallas_call(
        _dec_body,
        grid=(g,),
        in_specs=[
            pl.BlockSpec((bm, D_EMB_), lambda i: (i, 0)),
            pl.BlockSpec((bm, D_EMB_), lambda i: (i, 0)),
            const((D_EMB_, H2_)),
            const((1, H2_)),
            const((H2_, H1_)),
            const((1, H1_)),
            const((H1_, D_IN_)),
            const((1, D_IN_)),
        ],
        out_specs=[
            pl.BlockSpec((bm, D_EMB_), lambda i: (i, 0)),
            pl.BlockSpec((bm, D_IN_), lambda i: (i, 0)),
            pl.BlockSpec((1, 8, 128), lambda i: (i, 0, 0)),
        ],
        out_shape=[
            jax.ShapeDtypeStruct((z.shape[0], D_EMB_), jnp.float32),
            jax.ShapeDtypeStruct((z.shape[0], D_IN_), jnp.float32),
            jax.ShapeDtypeStruct((g, 8, 128), jnp.float32),
        ],
    )(z, q, Wd1b, bd1, Wd2b, bd2, Wd3b, bd3)


def kernel(x, We1, be1, We2, be2, We3, be3, codebook, Wd1, bd1, Wd2, bd2,
           Wd3, bd3):
    f32 = jnp.float32
    bf16 = jnp.bfloat16
    nc = 2
    cbT = codebook.T
    zs, idxs, qs = [], [], []
    for c in range(nc):
        xc = jax.lax.slice_in_dim(x, c * (B // nc), (c + 1) * (B // nc))
        zc, idx128c = _enc_vq(
            xc, We1, be1.reshape(1, -1), We2, be2.reshape(1, -1), We3,
            be3.reshape(1, -1), codebook, cbT)
        idxc = idx128c[:, 0]
        zs.append(zc)
        idxs.append(idxc)
        qs.append(_sc_gather(codebook, idxc.reshape(1, -1)))
    z = jnp.concatenate(zs, axis=0)
    idx = jnp.concatenate(idxs, axis=0)
    q = jnp.concatenate(qs, axis=0)
    zq, recon, lp = _decode(
        z, q, Wd1.astype(bf16), bd1.reshape(1, -1).astype(f32),
        Wd2.astype(bf16), bd2.reshape(1, -1).astype(f32),
        Wd3.astype(bf16), bd3.reshape(1, -1).astype(f32))
    m = jnp.sum(lp[:, 0, 0]) / (B * D_EMB_)
    vq_loss = m + 0.25 * m
    return recon, z, zq, vq_loss, idx


# speedup vs baseline: 1.2618x; 1.2618x over previous
import functools

import jax
import jax.numpy as jnp
from jax.experimental import pallas as pl
from jax.experimental.pallas import tpu as pltpu
from jax.experimental.pallas import tpu_sc as plsc

B, D_IN_ = 4096, 3072
H1_, H2_ = 1024, 512
D_EMB_, K_ = 256, 8192

_BM1 = 256
_BM3 = 256


def _rowsum_sq_lanes(v):
    sq = v * v
    s1 = sq[:, 0:128] + sq[:, 128:256]
    acc = s1[:, 0:8]
    for j in range(1, 16):
        acc = acc + s1[:, 8 * j:8 * (j + 1)]
    a4 = acc[:, 0:4] + acc[:, 4:8]
    a2 = a4[:, 0:2] + a4[:, 2:4]
    return a2[:, 0:1] + a2[:, 1:2]


def _rowsum_sq_sublanes(vT):
    sq = vT * vT
    s1 = sq[0:128, :] + sq[128:256, :]
    acc = s1[0:8, :]
    for j in range(1, 16):
        acc = acc + s1[8 * j:8 * (j + 1), :]
    a4 = acc[0:4, :] + acc[4:8, :]
    a2 = a4[0:2, :] + a4[2:4, :]
    return a2[0:1, :] + a2[1:2, :]


def _enc_vq_body(x_ref, we1_ref, be1_ref, we2_ref, be2_ref, we3_ref, be3_ref,
                 cb_ref, cbt_ref, z_ref, idx_ref, bv_ref):
    f32 = jnp.float32

    @pl.when(pl.program_id(0) == 0)
    def _():
        bv_ref[...] = _rowsum_sq_sublanes(cbt_ref[...])

    h1 = jnp.maximum(
        jax.lax.dot_general(x_ref[...], we1_ref[...], (((1,), (0,)), ((), ())),
                            preferred_element_type=f32) + be1_ref[...], 0.0)
    h2 = jnp.maximum(
        jax.lax.dot_general(h1, we2_ref[...], (((1,), (0,)), ((), ())),
                            preferred_element_type=f32) + be2_ref[...], 0.0)
    z = jax.lax.dot_general(h2, we3_ref[...], (((1,), (0,)), ((), ())),
                            preferred_element_type=f32) + be3_ref[...]
    a = _rowsum_sq_lanes(z)
    m2 = jax.lax.dot_general(z * 2.0, cb_ref[...], (((1,), (1,)), ((), ())),
                             preferred_element_type=f32)
    dist = (a + bv_ref[...]) - m2
    dmin = jnp.min(dist, axis=1, keepdims=True)
    iota = jax.lax.broadcasted_iota(jnp.int32, dist.shape, 1)
    idx = jnp.min(jnp.where(dist == dmin, iota, jnp.int32(K_)),
                  axis=1, keepdims=True)
    z_ref[...] = z
    idx_ref[...] = jnp.broadcast_to(idx, (idx.shape[0], 128))


def _enc_vq(x, We1, be1, We2, be2, We3, be3, cb, cbT):
    bm = _BM1
    grid = (x.shape[0] // bm,)
    const = lambda shape: pl.BlockSpec(shape, lambda i: (0,) * len(shape))
    return pl.pallas_call(
        _enc_vq_body,
        grid=grid,
        in_specs=[
            pl.BlockSpec((bm, D_IN_), lambda i: (i, 0)),
            const((D_IN_, H1_)),
            const((1, H1_)),
            const((H1_, H2_)),
            const((1, H2_)),
            const((H2_, D_EMB_)),
            const((1, D_EMB_)),
            const((K_, D_EMB_)),
            const((D_EMB_, K_)),
        ],
        out_specs=[
            pl.BlockSpec((bm, D_EMB_), lambda i: (i, 0)),
            pl.BlockSpec((bm, 128), lambda i: (i, 0)),
        ],
        out_shape=[
            jax.ShapeDtypeStruct((x.shape[0], D_EMB_), jnp.float32),
            jax.ShapeDtypeStruct((x.shape[0], 128), jnp.int32),
        ],
        scratch_shapes=[pltpu.VMEM((1, K_), jnp.float32)],
    )(x, We1, be1, We2, be2, We3, be3, cb, cbT)


def _sc_gather(cb, idx_row):
    mesh = plsc.VectorSubcoreMesh(core_axis_name="c", subcore_axis_name="s")
    win = 128
    n = idx_row.shape[1]

    @functools.partial(
        pl.kernel,
        out_type=jax.ShapeDtypeStruct((n, cb.shape[1]), cb.dtype),
        mesh=mesh)
    def k(cb_hbm, i_hbm, o_hbm):
        def body(i_vmem, o_vmem):
            pltpu.sync_copy(cb_hbm.at[i_vmem.at[0]], o_vmem)

        pltpu.emit_pipeline(
            body,
            grid=(n // win,),
            in_specs=[pl.BlockSpec((1, win), index_map=lambda i: (0, i))],
            out_specs=[pl.BlockSpec((win, cb.shape[1]),
                                    index_map=lambda i: (i, 0))],
            core_axis_name=("c", "s"),
            dimension_semantics=(pltpu.PARALLEL,),
        )(i_hbm, o_hbm)

    return k(cb, idx_row)


def _dec_body(z_ref, q_ref, wd1_ref, bd1_ref, wd2_ref, bd2_ref, wd3_ref,
              bd3_ref, zq_ref, recon_ref, lp_ref, w1_ref, w2_ref, w3_ref):
    f32 = jnp.float32
    bf16 = jnp.bfloat16

    @pl.when(pl.program_id(0) == 0)
    def _():
        w1_ref[...] = wd1_ref[...].astype(bf16)
        w2_ref[...] = wd2_ref[...].astype(bf16)
        w3_ref[...] = wd3_ref[...].astype(bf16)

    z = z_ref[...]
    q = q_ref[...]
    qz = q - z
    zq = z + qz
    lp = jnp.sum(qz * qz)
    d1 = jnp.maximum(
        jax.lax.dot_general(zq.astype(bf16), w1_ref[...],
                            (((1,), (0,)), ((), ())),
                            preferred_element_type=f32) + bd1_ref[...], 0.0)
    d2 = jnp.maximum(
        jax.lax.dot_general(d1.astype(bf16), w2_ref[...],
                            (((1,), (0,)), ((), ())),
                            preferred_element_type=f32) + bd2_ref[...], 0.0)
    logits = jax.lax.dot_general(d2.astype(bf16), w3_ref[...],
                                 (((1,), (0,)), ((), ())),
                                 preferred_element_type=f32) + bd3_ref[...]
    zq_ref[...] = zq
    recon_ref[...] = jax.nn.sigmoid(logits)
    lp_ref[...] = jnp.broadcast_to(lp.reshape(1, 1, 1), (1, 8, 128))


def _decode(z, q, Wd1b, bd1, Wd2b, bd2, Wd3b, bd3):
    bm = _BM3
    g = z.shape[0] // bm
    const = lambda shape: pl.BlockSpec(shape, lambda i: (0,) * len(shape))
    return pl.pallas_call(
        _dec_body,
        grid=(g,),
        in_specs=[
            pl.BlockSpec((bm, D_EMB_), lambda i: (i, 0)),
            pl.BlockSpec((bm, D_EMB_), lambda i: (i, 0)),
            const((D_EMB_, H2_)),
            const((1, H2_)),
            const((H2_, H1_)),
            const((1, H1_)),
            const((H1_, D_IN_)),
            const((1, D_IN_)),
        ],
        out_specs=[
            pl.BlockSpec((bm, D_EMB_), lambda i: (i, 0)),
            pl.BlockSpec((bm, D_IN_), lambda i: (i, 0)),
            pl.BlockSpec((1, 8, 128), lambda i: (i, 0, 0)),
        ],
        out_shape=[
            jax.ShapeDtypeStruct((z.shape[0], D_EMB_), jnp.float32),
            jax.ShapeDtypeStruct((z.shape[0], D_IN_), jnp.float32),
            jax.ShapeDtypeStruct((g, 8, 128), jnp.float32),
        ],
        scratch_shapes=[
            pltpu.VMEM((D_EMB_, H2_), jnp.bfloat16),
            pltpu.VMEM((H2_, H1_), jnp.bfloat16),
            pltpu.VMEM((H1_, D_IN_), jnp.bfloat16),
        ],
    )(z, q, Wd1b, bd1, Wd2b, bd2, Wd3b, bd3)


def kernel(x, We1, be1, We2, be2, We3, be3, codebook, Wd1, bd1, Wd2, bd2,
           Wd3, bd3):
    f32 = jnp.float32
    bf16 = jnp.bfloat16
    z, idx128 = _enc_vq(
        x, We1, be1.reshape(1, -1), We2, be2.reshape(1, -1), We3,
        be3.reshape(1, -1), codebook, codebook.T)
    idx = idx128[:, 0]
    q = _sc_gather(codebook, idx.reshape(1, -1))
    zq, recon, lp = _decode(
        z, q, Wd1, bd1.reshape(1, -1), Wd2, bd2.reshape(1, -1),
        Wd3, bd3.reshape(1, -1))
    m = jnp.sum(lp[:, 0, 0]) / (B * D_EMB_)
    vq_loss = m + 0.25 * m
    return recon, z, zq, vq_loss, idx


# speedup vs baseline: 1.3163x; 1.0432x over previous
import functools

import jax
import jax.numpy as jnp
from jax.experimental import pallas as pl
from jax.experimental.pallas import tpu as pltpu
from jax.experimental.pallas import tpu_sc as plsc

B, D_IN_ = 4096, 3072
H1_, H2_ = 1024, 512
D_EMB_, K_ = 256, 8192

_BM1 = 256
_BM3 = 256


def _rowsum_sq_lanes(v):
    sq = v * v
    s1 = sq[:, 0:128] + sq[:, 128:256]
    acc = s1[:, 0:8]
    for j in range(1, 16):
        acc = acc + s1[:, 8 * j:8 * (j + 1)]
    a4 = acc[:, 0:4] + acc[:, 4:8]
    a2 = a4[:, 0:2] + a4[:, 2:4]
    return a2[:, 0:1] + a2[:, 1:2]


def _rowsum_sq_sublanes(vT):
    sq = vT * vT
    s1 = sq[0:128, :] + sq[128:256, :]
    acc = s1[0:8, :]
    for j in range(1, 16):
        acc = acc + s1[8 * j:8 * (j + 1), :]
    a4 = acc[0:4, :] + acc[4:8, :]
    a2 = a4[0:2, :] + a4[2:4, :]
    return a2[0:1, :] + a2[1:2, :]


def _enc_vq_body(x_ref, we1_ref, be1_ref, we2_ref, be2_ref, we3_ref, be3_ref,
                 cb_ref, cbt_ref, z_ref, idx_ref, bv_ref):
    f32 = jnp.float32

    @pl.when(pl.program_id(0) == 0)
    def _():
        bv_ref[...] = _rowsum_sq_sublanes(cbt_ref[...])

    h1 = jnp.maximum(
        jax.lax.dot_general(x_ref[...], we1_ref[...], (((1,), (0,)), ((), ())),
                            preferred_element_type=f32) + be1_ref[...], 0.0)
    h2 = jnp.maximum(
        jax.lax.dot_general(h1, we2_ref[...], (((1,), (0,)), ((), ())),
                            preferred_element_type=f32) + be2_ref[...], 0.0)
    z = jax.lax.dot_general(h2, we3_ref[...], (((1,), (0,)), ((), ())),
                            preferred_element_type=f32) + be3_ref[...]
    a = _rowsum_sq_lanes(z)
    m2 = jax.lax.dot_general(z * 2.0, cb_ref[...], (((1,), (1,)), ((), ())),
                             preferred_element_type=f32)
    dist = (a + bv_ref[...]) - m2
    idx = jnp.argmin(dist, axis=1).reshape(-1, 1).astype(jnp.int32)
    z_ref[...] = z
    idx_ref[...] = jnp.broadcast_to(idx, (idx.shape[0], 128))


def _enc_vq(x, We1, be1, We2, be2, We3, be3, cb, cbT):
    bm = _BM1
    grid = (x.shape[0] // bm,)
    const = lambda shape: pl.BlockSpec(shape, lambda i: (0,) * len(shape))
    return pl.pallas_call(
        _enc_vq_body,
        grid=grid,
        in_specs=[
            pl.BlockSpec((bm, D_IN_), lambda i: (i, 0)),
            const((D_IN_, H1_)),
            const((1, H1_)),
            const((H1_, H2_)),
            const((1, H2_)),
            const((H2_, D_EMB_)),
            const((1, D_EMB_)),
            const((K_, D_EMB_)),
            const((D_EMB_, K_)),
        ],
        out_specs=[
            pl.BlockSpec((bm, D_EMB_), lambda i: (i, 0)),
            pl.BlockSpec((bm, 128), lambda i: (i, 0)),
        ],
        out_shape=[
            jax.ShapeDtypeStruct((x.shape[0], D_EMB_), jnp.float32),
            jax.ShapeDtypeStruct((x.shape[0], 128), jnp.int32),
        ],
        scratch_shapes=[pltpu.VMEM((1, K_), jnp.float32)],
    )(x, We1, be1, We2, be2, We3, be3, cb, cbT)


def _sc_gather(cb, idx_row):
    mesh = plsc.VectorSubcoreMesh(core_axis_name="c", subcore_axis_name="s")
    win = 128
    n = idx_row.shape[1]

    @functools.partial(
        pl.kernel,
        out_type=jax.ShapeDtypeStruct((n, cb.shape[1]), cb.dtype),
        mesh=mesh)
    def k(cb_hbm, i_hbm, o_hbm):
        def body(i_vmem, o_vmem):
            pltpu.sync_copy(cb_hbm.at[i_vmem.at[0]], o_vmem)

        pltpu.emit_pipeline(
            body,
            grid=(n // win,),
            in_specs=[pl.BlockSpec((1, win), index_map=lambda i: (0, i))],
            out_specs=[pl.BlockSpec((win, cb.shape[1]),
                                    index_map=lambda i: (i, 0))],
            core_axis_name=("c", "s"),
            dimension_semantics=(pltpu.PARALLEL,),
        )(i_hbm, o_hbm)

    return k(cb, idx_row)


def _dec_body(z_ref, q_ref, wd1_ref, bd1_ref, wd2_ref, bd2_ref, wd3_ref,
              bd3_ref, zq_ref, recon_ref, lp_ref, w1_ref, w2_ref, w3_ref):
    f32 = jnp.float32
    bf16 = jnp.bfloat16

    @pl.when(pl.program_id(0) == 0)
    def _():
        w1_ref[...] = wd1_ref[...].astype(bf16)
        w2_ref[...] = wd2_ref[...].astype(bf16)
        w3_ref[...] = wd3_ref[...].astype(bf16)

    z = z_ref[...]
    q = q_ref[...]
    qz = q - z
    zq = z + qz
    lp = jnp.sum(qz * qz)
    d1 = jnp.maximum(
        jax.lax.dot_general(zq.astype(bf16), w1_ref[...],
                            (((1,), (0,)), ((), ())),
                            preferred_element_type=f32) + bd1_ref[...], 0.0)
    d2 = jnp.maximum(
        jax.lax.dot_general(d1.astype(bf16), w2_ref[...],
                            (((1,), (0,)), ((), ())),
                            preferred_element_type=f32) + bd2_ref[...], 0.0)
    logits = jax.lax.dot_general(d2.astype(bf16), w3_ref[...],
                                 (((1,), (0,)), ((), ())),
                                 preferred_element_type=f32) + bd3_ref[...]
    zq_ref[...] = zq
    recon_ref[...] = jax.nn.sigmoid(logits)
    lp_ref[...] = jnp.broadcast_to(lp.reshape(1, 1, 1), (1, 8, 128))


def _decode(z, q, Wd1b, bd1, Wd2b, bd2, Wd3b, bd3):
    bm = _BM3
    g = z.shape[0] // bm
    const = lambda shape: pl.BlockSpec(shape, lambda i: (0,) * len(shape))
    return pl.pallas_call(
        _dec_body,
        grid=(g,),
        in_specs=[
            pl.BlockSpec((bm, D_EMB_), lambda i: (i, 0)),
            pl.BlockSpec((bm, D_EMB_), lambda i: (i, 0)),
            const((D_EMB_, H2_)),
            const((1, H2_)),
            const((H2_, H1_)),
            const((1, H1_)),
            const((H1_, D_IN_)),
            const((1, D_IN_)),
        ],
        out_specs=[
            pl.BlockSpec((bm, D_EMB_), lambda i: (i, 0)),
            pl.BlockSpec((bm, D_IN_), lambda i: (i, 0)),
            pl.BlockSpec((1, 8, 128), lambda i: (i, 0, 0)),
        ],
        out_shape=[
            jax.ShapeDtypeStruct((z.shape[0], D_EMB_), jnp.float32),
            jax.ShapeDtypeStruct((z.shape[0], D_IN_), jnp.float32),
            jax.ShapeDtypeStruct((g, 8, 128), jnp.float32),
        ],
        scratch_shapes=[
            pltpu.VMEM((D_EMB_, H2_), jnp.bfloat16),
            pltpu.VMEM((H2_, H1_), jnp.bfloat16),
            pltpu.VMEM((H1_, D_IN_), jnp.bfloat16),
        ],
    )(z, q, Wd1b, bd1, Wd2b, bd2, Wd3b, bd3)


def kernel(x, We1, be1, We2, be2, We3, be3, codebook, Wd1, bd1, Wd2, bd2,
           Wd3, bd3):
    f32 = jnp.float32
    bf16 = jnp.bfloat16
    z, idx128 = _enc_vq(
        x, We1, be1.reshape(1, -1), We2, be2.reshape(1, -1), We3,
        be3.reshape(1, -1), codebook, codebook.T)
    idx = idx128[:, 0]
    q = _sc_gather(codebook, idx.reshape(1, -1))
    zq, recon, lp = _decode(
        z, q, Wd1, bd1.reshape(1, -1), Wd2, bd2.reshape(1, -1),
        Wd3, bd3.reshape(1, -1))
    m = jnp.sum(lp[:, 0, 0]) / (B * D_EMB_)
    vq_loss = m + 0.25 * m
    return recon, z, zq, vq_loss, idx
